# baseline (speedup 1.0000x reference)
import functools

import jax
import jax.numpy as jnp
from jax import lax
from jax.experimental import pallas as pl
from jax.experimental.pallas import tpu as pltpu
from jax.experimental.pallas import tpu_sc as plsc

RING = 128
DRAIN_AT = RING - 16


def _make_sc_gather(V, D, B):
    info = plsc.get_sparse_core_info()
    NC, NS = info.num_cores, info.num_subcores
    NW = NC * NS
    assert D == 64 and B % 16 == 0
    n_strips_all = (V + 127) // 128
    last_strip = V // 128
    strips_base = n_strips_all // NW
    strips_rem = n_strips_all % NW
    n_vreg_idx = B // 16
    ns_static = strips_base + 2 - ((strips_base + 2) % 2)
    smax = strips_base + 1
    mesh = plsc.VectorSubcoreMesh(core_axis_name="c", subcore_axis_name="s")

    @functools.partial(
        pl.kernel,
        mesh=mesh,
        out_type=jax.ShapeDtypeStruct((B, D), jnp.float32),
        scratch_types=[
            pltpu.VMEM((B,), jnp.int32),
            pltpu.VMEM((B + 16,), jnp.int32),
            pltpu.VMEM((B + 16,), jnp.int32),
            pltpu.VMEM((B + 16,), jnp.int32),
            pltpu.VMEM((B + 16,), jnp.int32),
            pltpu.VMEM((2, 64, 128), jnp.float32),
            pltpu.VMEM((RING, 64), jnp.float32),
            pltpu.SMEM((smax + 2,), jnp.int32),
            pltpu.SMEM((smax + 2,), jnp.int32),
            pltpu.SemaphoreType.DMA,
            pltpu.SemaphoreType.DMA,
            pltpu.SemaphoreType.DMA,
        ],
        compiler_params=pltpu.CompilerParams(needs_layout_passes=False),
    )
    def k(idx_hbm, tableT_hbm, lastT_hbm, out_hbm, idx_v, pi_v, pj_v, qi_v,
          qj_v, strips, ring2, bins_s, offs_s, sem_a, sem_b, sem_out):
        w = lax.axis_index("s") * NC + lax.axis_index("c")
        start = w * strips_base + jnp.minimum(w, strips_rem)
        cnt = strips_base + (w < strips_rem).astype(jnp.int32)
        end = start + cnt
        trash = jnp.int32(smax)

        pltpu.sync_copy(idx_hbm, idx_v)

        lane = lax.iota(jnp.int32, 16)
        neg1 = jnp.full((16,), -1, jnp.int32)

        def collect(g, ptr):
            v = idx_v[pl.ds(g * 16, 16)]
            tcv = lax.shift_right_logical(v, 7)
            m = jnp.logical_and(tcv >= start, tcv < end)
            nm = plsc.all_reduce_population_count(m)[0]

            def append(ptr):
                cums = plsc.cumsum(m.astype(jnp.int32))
                pos = cums + (ptr - 1)
                plsc.store_scatter(pi_v, [pos], v, mask=m)
                plsc.store_scatter(pj_v, [pos], g * 16 + lane, mask=m)
                return ptr + nm

            return lax.cond(nm > 0, append, lambda ptr: ptr, ptr)

        n_w = lax.fori_loop(0, n_vreg_idx, collect, jnp.int32(0))
        n_pv = (n_w + 15) // 16
        plsc.store_scatter(pi_v, [n_w + lane], neg1)

        def zero_hist(s, c):
            offs_s[s] = 0
            return c

        lax.fori_loop(0, smax + 2, zero_hist, 0)

        def hist(p, c):
            iv = pi_v[pl.ds(p * 16, 16)]
            sv = lax.shift_right_logical(iv, 7) - start
            sv = jnp.clip(sv, 0, trash)
            for l in range(16):
                s = sv[l]
                offs_s[s] = offs_s[s] + 1
            return c

        lax.fori_loop(0, n_pv, hist, 0)

        def prefix(s, run):
            h = offs_s[s]
            bins_s[s] = run
            offs_s[s] = run
            return run + h

        lax.fori_loop(0, smax + 2, prefix, jnp.int32(0))

        def place(p, c):
            iv = pi_v[pl.ds(p * 16, 16)]
            jv = pj_v[pl.ds(p * 16, 16)]
            sv = lax.shift_right_logical(iv, 7) - start
            sv = jnp.clip(sv, 0, trash)
            for l in range(16):
                s = sv[l]
                pos = offs_s[s]
                offs_s[s] = pos + 1
                posv = jnp.full((16,), pos, jnp.int32)
                onehot = lane == l
                plsc.store_scatter(qi_v, [posv], iv, mask=onehot)
                plsc.store_scatter(qj_v, [posv], jv, mask=onehot)
            return c

        lax.fori_loop(0, n_pv, place, 0)

        def strip_needed(tc):
            s_loc = tc - start
            sc = jnp.clip(s_loc, 0, smax)
            nonempty = bins_s[sc + 1] > bins_s[sc]
            return jnp.logical_and(
                jnp.logical_and(s_loc >= 0, s_loc < cnt), nonempty
            )

        def fire(tc, buf, sem):
            needed = strip_needed(tc)

            @pl.when(jnp.logical_and(needed, tc < last_strip))
            def _():
                off = pl.multiple_of(tc * 128, 128)
                pltpu.async_copy(
                    tableT_hbm.at[:, pl.ds(off, 128)], strips.at[buf], sem
                )

            @pl.when(jnp.logical_and(needed, tc == last_strip))
            def _():
                pltpu.async_copy(lastT_hbm, strips.at[buf], sem)

        def wait(tc, buf, sem):
            @pl.when(strip_needed(tc))
            def _():
                pltpu.make_async_copy(
                    tableT_hbm.at[:, pl.ds(0, 128)], strips.at[buf], sem
                ).wait()

        def drain_all(o):
            def dw(_, c):
                pltpu.make_async_copy(
                    ring2.at[pl.ds(0, 1), :], out_hbm.at[pl.ds(0, 1), :],
                    sem_out,
                ).wait()
                return c

            lax.fori_loop(0, o, dw, 0)
            return jnp.int32(0)

        def scan_strip(tc, buf, qo):
            s_loc = tc - start

            def go(qo):
                b0 = bins_s[s_loc]
                b1 = bins_s[s_loc + 1]
                bufv = jnp.full((16,), buf, jnp.int32)

                def one(t, qo):
                    q, o = qo
                    pv = jnp.full((16,), b0 + t, jnp.int32)
                    i = plsc.load_gather(qi_v, [pv])[0]
                    j = plsc.load_gather(qj_v, [pv])[0]
                    col = jnp.bitwise_and(i, 127)
                    colv = jnp.full((16,), col, jnp.int32)
                    slot = lax.rem(q, jnp.int32(RING))
                    slotv = jnp.full((16,), slot, jnp.int32)
                    for g4 in range(4):
                        vals = plsc.load_gather(
                            strips, [bufv, g4 * 16 + lane, colv]
                        )
                        plsc.store_scatter(
                            ring2, [slotv, g4 * 16 + lane], vals
                        )
                    pltpu.async_copy(
                        ring2.at[pl.ds(slot, 1), :],
                        out_hbm.at[pl.ds(j, 1), :],
                        sem_out,
                    )
                    o = o + 1
                    o = lax.cond(o >= DRAIN_AT, drain_all, lambda o: o, o)
                    return (q + 1, o)

                return lax.fori_loop(0, b1 - b0, one, qo)

            in_range = jnp.logical_and(s_loc >= 0, s_loc < cnt)
            return lax.cond(in_range, go, lambda qo: qo, qo)

        fire(start, 0, sem_a)
        qo = (jnp.int32(0), jnp.int32(0))

        def two_strips(ss, qo):
            t0 = start + 2 * ss
            t1 = t0 + 1
            t2 = t0 + 2
            fire(t1, 1, sem_b)
            wait(t0, 0, sem_a)
            qo = scan_strip(t0, 0, qo)
            fire(t2, 0, sem_a)
            wait(t1, 1, sem_b)
            qo = scan_strip(t1, 1, qo)
            return qo

        qo = lax.fori_loop(0, ns_static // 2, two_strips, qo)
        wait(start + ns_static, 0, sem_a)
        q, o = qo
        drain_all(o)

    return k


@jax.jit
def kernel(class_labels, embedding_weight):
    (B,) = class_labels.shape
    V, D = embedding_weight.shape
    k = _make_sc_gather(V, D, B)
    n_last = V % 128 if V % 128 else 128
    lastT = jnp.pad(
        embedding_weight[V - n_last :, :].T, ((0, 0), (0, 128 - n_last))
    )
    return k(class_labels.astype(jnp.int32), embedding_weight.T, lastT)


# speedup vs baseline: 1.0627x; 1.0627x over previous
import functools

import jax
import jax.numpy as jnp
from jax import lax
from jax.experimental import pallas as pl
from jax.experimental.pallas import tpu as pltpu
from jax.experimental.pallas import tpu_sc as plsc

RING = 128
DRAIN_AT = RING - 16


def _make_sc_gather(V, D, B):
    info = plsc.get_sparse_core_info()
    NC, NS = info.num_cores, info.num_subcores
    NW = NC * NS
    assert D == 64 and B % 16 == 0
    n_strips_all = (V + 127) // 128
    last_strip = V // 128
    strips_base = n_strips_all // NW
    strips_rem = n_strips_all % NW
    n_vreg_idx = B // 16
    ns_static = strips_base + 2 - ((strips_base + 2) % 2)
    smax = strips_base + 1
    mesh = plsc.VectorSubcoreMesh(core_axis_name="c", subcore_axis_name="s")

    @functools.partial(
        pl.kernel,
        mesh=mesh,
        out_type=jax.ShapeDtypeStruct((B, D), jnp.float32),
        scratch_types=[
            pltpu.VMEM((B,), jnp.int32),
            pltpu.VMEM((B + 16,), jnp.int32),
            pltpu.VMEM((B + 16,), jnp.int32),
            pltpu.VMEM((B + 16,), jnp.int32),
            pltpu.VMEM((B + 16,), jnp.int32),
            pltpu.VMEM((2, 64, 128), jnp.float32),
            pltpu.VMEM((RING, 64), jnp.float32),
            pltpu.SMEM((smax + 2,), jnp.int32),
            pltpu.SMEM((smax + 2,), jnp.int32),
            pltpu.SemaphoreType.DMA,
            pltpu.SemaphoreType.DMA,
            pltpu.SemaphoreType.DMA,
        ],
        compiler_params=pltpu.CompilerParams(needs_layout_passes=False),
    )
    def k(idx_hbm, tableT_hbm, lastT_hbm, out_hbm, idx_v, pi_v, pj_v, qi_v,
          qj_v, strips, ring2, bins_s, offs_s, sem_a, sem_b, sem_out):
        w = lax.axis_index("s") * NC + lax.axis_index("c")
        start = w * strips_base + jnp.minimum(w, strips_rem)
        cnt = strips_base + (w < strips_rem).astype(jnp.int32)
        end = start + cnt
        trash = jnp.int32(smax)

        pltpu.sync_copy(idx_hbm, idx_v)

        lane = lax.iota(jnp.int32, 16)
        neg1 = jnp.full((16,), -1, jnp.int32)

        def collect(g, ptr):
            v = idx_v[pl.ds(g * 16, 16)]
            tcv = lax.shift_right_logical(v, 7)
            m = jnp.logical_and(tcv >= start, tcv < end)
            cums = plsc.cumsum(m.astype(jnp.int32))
            pos = cums + (ptr - 1)
            plsc.store_scatter(pi_v, [pos], v, mask=m)
            plsc.store_scatter(pj_v, [pos], g * 16 + lane, mask=m)
            return ptr + cums[15]

        n_w = lax.fori_loop(0, n_vreg_idx, collect, jnp.int32(0))
        n_pv = (n_w + 15) // 16
        plsc.store_scatter(pi_v, [n_w + lane], neg1)

        def zero_hist(s, c):
            offs_s[s] = 0
            return c

        lax.fori_loop(0, smax + 2, zero_hist, 0)

        def hist(p, c):
            iv = pi_v[pl.ds(p * 16, 16)]
            sv = lax.shift_right_logical(iv, 7) - start
            sv = jnp.clip(sv, 0, trash)
            for l in range(16):
                s = sv[l]
                offs_s[s] = offs_s[s] + 1
            return c

        lax.fori_loop(0, n_pv, hist, 0)

        def prefix(s, run):
            h = offs_s[s]
            bins_s[s] = run
            offs_s[s] = run
            return run + h

        lax.fori_loop(0, smax + 2, prefix, jnp.int32(0))

        def place(p, c):
            iv = pi_v[pl.ds(p * 16, 16)]
            jv = pj_v[pl.ds(p * 16, 16)]
            sv = lax.shift_right_logical(iv, 7) - start
            sv = jnp.clip(sv, 0, trash)
            for l in range(16):
                s = sv[l]
                pos = offs_s[s]
                offs_s[s] = pos + 1
                posv = jnp.full((16,), pos, jnp.int32)
                onehot = lane == l
                plsc.store_scatter(qi_v, [posv], iv, mask=onehot)
                plsc.store_scatter(qj_v, [posv], jv, mask=onehot)
            return c

        lax.fori_loop(0, n_pv, place, 0)

        def strip_needed(tc):
            s_loc = tc - start
            sc = jnp.clip(s_loc, 0, smax)
            nonempty = bins_s[sc + 1] > bins_s[sc]
            return jnp.logical_and(
                jnp.logical_and(s_loc >= 0, s_loc < cnt), nonempty
            )

        def fire(tc, buf, sem):
            needed = strip_needed(tc)

            @pl.when(jnp.logical_and(needed, tc < last_strip))
            def _():
                off = pl.multiple_of(tc * 128, 128)
                pltpu.async_copy(
                    tableT_hbm.at[:, pl.ds(off, 128)], strips.at[buf], sem
                )

            @pl.when(jnp.logical_and(needed, tc == last_strip))
            def _():
                pltpu.async_copy(lastT_hbm, strips.at[buf], sem)

        def wait(tc, buf, sem):
            @pl.when(strip_needed(tc))
            def _():
                pltpu.make_async_copy(
                    tableT_hbm.at[:, pl.ds(0, 128)], strips.at[buf], sem
                ).wait()

        def drain_all(o):
            def dw(_, c):
                pltpu.make_async_copy(
                    ring2.at[pl.ds(0, 1), :], out_hbm.at[pl.ds(0, 1), :],
                    sem_out,
                ).wait()
                return c

            lax.fori_loop(0, o, dw, 0)
            return jnp.int32(0)

        def scan_strip(tc, buf, qo):
            s_loc = tc - start

            def go(qo):
                b0 = bins_s[s_loc]
                b1 = bins_s[s_loc + 1]
                bufv = jnp.full((16,), buf, jnp.int32)

                def one(t, qo):
                    q, o = qo
                    pv = jnp.full((16,), b0 + t, jnp.int32)
                    i = plsc.load_gather(qi_v, [pv])[0]
                    j = plsc.load_gather(qj_v, [pv])[0]
                    col = jnp.bitwise_and(i, 127)
                    colv = jnp.full((16,), col, jnp.int32)
                    slot = lax.rem(q, jnp.int32(RING))
                    slotv = jnp.full((16,), slot, jnp.int32)
                    for g4 in range(4):
                        vals = plsc.load_gather(
                            strips, [bufv, g4 * 16 + lane, colv]
                        )
                        plsc.store_scatter(
                            ring2, [slotv, g4 * 16 + lane], vals
                        )
                    pltpu.async_copy(
                        ring2.at[pl.ds(slot, 1), :],
                        out_hbm.at[pl.ds(j, 1), :],
                        sem_out,
                    )
                    o = o + 1
                    o = lax.cond(o >= DRAIN_AT, drain_all, lambda o: o, o)
                    return (q + 1, o)

                return lax.fori_loop(0, b1 - b0, one, qo)

            in_range = jnp.logical_and(s_loc >= 0, s_loc < cnt)
            return lax.cond(in_range, go, lambda qo: qo, qo)

        fire(start, 0, sem_a)
        qo = (jnp.int32(0), jnp.int32(0))

        def two_strips(ss, qo):
            t0 = start + 2 * ss
            t1 = t0 + 1
            t2 = t0 + 2
            fire(t1, 1, sem_b)
            wait(t0, 0, sem_a)
            qo = scan_strip(t0, 0, qo)
            fire(t2, 0, sem_a)
            wait(t1, 1, sem_b)
            qo = scan_strip(t1, 1, qo)
            return qo

        qo = lax.fori_loop(0, ns_static // 2, two_strips, qo)
        wait(start + ns_static, 0, sem_a)
        q, o = qo
        drain_all(o)

    return k


@jax.jit
def kernel(class_labels, embedding_weight):
    (B,) = class_labels.shape
    V, D = embedding_weight.shape
    k = _make_sc_gather(V, D, B)
    n_last = V % 128 if V % 128 else 128
    lastT = jnp.pad(
        embedding_weight[V - n_last :, :].T, ((0, 0), (0, 128 - n_last))
    )
    return k(class_labels.astype(jnp.int32), embedding_weight.T, lastT)


# speedup vs baseline: 1.1148x; 1.0490x over previous
import functools

import jax
import jax.numpy as jnp
from jax import lax
from jax.experimental import pallas as pl
from jax.experimental.pallas import tpu as pltpu
from jax.experimental.pallas import tpu_sc as plsc

RING = 128
DRAIN_AT = RING - 16


def _make_sc_gather(V, D, B):
    info = plsc.get_sparse_core_info()
    NC, NS = info.num_cores, info.num_subcores
    NW = NC * NS
    assert D == 64 and B % 16 == 0
    n_strips_all = (V + 127) // 128
    last_strip = V // 128
    strips_base = n_strips_all // NW
    strips_rem = n_strips_all % NW
    n_vreg_idx = B // 16
    ns_static = strips_base + 2 - ((strips_base + 2) % 2)
    smax = strips_base + 1
    mesh = plsc.VectorSubcoreMesh(core_axis_name="c", subcore_axis_name="s")

    @functools.partial(
        pl.kernel,
        mesh=mesh,
        out_type=jax.ShapeDtypeStruct((B, D), jnp.float32),
        scratch_types=[
            pltpu.VMEM((B,), jnp.int32),
            pltpu.VMEM((B + 16,), jnp.int32),
            pltpu.VMEM((B + 16,), jnp.int32),
            pltpu.VMEM((B + 16,), jnp.int32),
            pltpu.VMEM((B + 16,), jnp.int32),
            pltpu.VMEM((2, 64, 128), jnp.float32),
            pltpu.VMEM((RING, 64), jnp.float32),
            pltpu.SMEM((smax + 2,), jnp.int32),
            pltpu.SMEM((smax + 2,), jnp.int32),
            pltpu.SemaphoreType.DMA,
            pltpu.SemaphoreType.DMA,
            pltpu.SemaphoreType.DMA,
        ],
        compiler_params=pltpu.CompilerParams(needs_layout_passes=False),
    )
    def k(idx_hbm, tableT_hbm, lastT_hbm, out_hbm, idx_v, pi_v, pj_v, qi_v,
          qj_v, strips, ring2, bins_s, offs_s, sem_a, sem_b, sem_out):
        w = lax.axis_index("s") * NC + lax.axis_index("c")
        start = w * strips_base + jnp.minimum(w, strips_rem)
        cnt = strips_base + (w < strips_rem).astype(jnp.int32)
        end = start + cnt
        trash = jnp.int32(smax)

        pltpu.sync_copy(idx_hbm, idx_v)

        lane = lax.iota(jnp.int32, 16)
        neg1 = jnp.full((16,), -1, jnp.int32)

        @plsc.parallel_loop(0, n_vreg_idx, unroll=4, carry=jnp.int32(0))
        def collect(g, ptr):
            v = idx_v[pl.ds(g * 16, 16)]
            tcv = lax.shift_right_logical(v, 7)
            m = jnp.logical_and(tcv >= start, tcv < end)
            cums = plsc.cumsum(m.astype(jnp.int32))
            pos = cums + (ptr - 1)
            plsc.store_scatter(pi_v, [pos], v, mask=m)
            plsc.store_scatter(pj_v, [pos], g * 16 + lane, mask=m)
            return ptr + cums[15]

        n_w = collect
        n_pv = (n_w + 15) // 16
        plsc.store_scatter(pi_v, [n_w + lane], neg1)

        def zero_hist(s, c):
            offs_s[s] = 0
            return c

        lax.fori_loop(0, smax + 2, zero_hist, 0)

        def hist(p, c):
            iv = pi_v[pl.ds(p * 16, 16)]
            sv = lax.shift_right_logical(iv, 7) - start
            sv = jnp.clip(sv, 0, trash)
            for l in range(16):
                s = sv[l]
                offs_s[s] = offs_s[s] + 1
            return c

        lax.fori_loop(0, n_pv, hist, 0)

        def prefix(s, run):
            h = offs_s[s]
            bins_s[s] = run
            offs_s[s] = run
            return run + h

        lax.fori_loop(0, smax + 2, prefix, jnp.int32(0))

        def place(p, c):
            iv = pi_v[pl.ds(p * 16, 16)]
            jv = pj_v[pl.ds(p * 16, 16)]
            sv = lax.shift_right_logical(iv, 7) - start
            sv = jnp.clip(sv, 0, trash)
            for l in range(16):
                s = sv[l]
                pos = offs_s[s]
                offs_s[s] = pos + 1
                posv = jnp.full((16,), pos, jnp.int32)
                onehot = lane == l
                plsc.store_scatter(qi_v, [posv], iv, mask=onehot)
                plsc.store_scatter(qj_v, [posv], jv, mask=onehot)
            return c

        lax.fori_loop(0, n_pv, place, 0)

        def strip_needed(tc):
            s_loc = tc - start
            sc = jnp.clip(s_loc, 0, smax)
            nonempty = bins_s[sc + 1] > bins_s[sc]
            return jnp.logical_and(
                jnp.logical_and(s_loc >= 0, s_loc < cnt), nonempty
            )

        def fire(tc, buf, sem):
            needed = strip_needed(tc)

            @pl.when(jnp.logical_and(needed, tc < last_strip))
            def _():
                off = pl.multiple_of(tc * 128, 128)
                pltpu.async_copy(
                    tableT_hbm.at[:, pl.ds(off, 128)], strips.at[buf], sem
                )

            @pl.when(jnp.logical_and(needed, tc == last_strip))
            def _():
                pltpu.async_copy(lastT_hbm, strips.at[buf], sem)

        def wait(tc, buf, sem):
            @pl.when(strip_needed(tc))
            def _():
                pltpu.make_async_copy(
                    tableT_hbm.at[:, pl.ds(0, 128)], strips.at[buf], sem
                ).wait()

        def drain_all(o):
            def dw(_, c):
                pltpu.make_async_copy(
                    ring2.at[pl.ds(0, 1), :], out_hbm.at[pl.ds(0, 1), :],
                    sem_out,
                ).wait()
                return c

            lax.fori_loop(0, o, dw, 0)
            return jnp.int32(0)

        def scan_strip(tc, buf, qo):
            s_loc = tc - start

            def go(qo):
                b0 = bins_s[s_loc]
                b1 = bins_s[s_loc + 1]
                bufv = jnp.full((16,), buf, jnp.int32)

                def one(t, qo):
                    q, o = qo
                    pv = jnp.full((16,), b0 + t, jnp.int32)
                    i = plsc.load_gather(qi_v, [pv])[0]
                    j = plsc.load_gather(qj_v, [pv])[0]
                    col = jnp.bitwise_and(i, 127)
                    colv = jnp.full((16,), col, jnp.int32)
                    slot = lax.rem(q, jnp.int32(RING))
                    slotv = jnp.full((16,), slot, jnp.int32)
                    for g4 in range(4):
                        vals = plsc.load_gather(
                            strips, [bufv, g4 * 16 + lane, colv]
                        )
                        plsc.store_scatter(
                            ring2, [slotv, g4 * 16 + lane], vals
                        )
                    pltpu.async_copy(
                        ring2.at[pl.ds(slot, 1), :],
                        out_hbm.at[pl.ds(j, 1), :],
                        sem_out,
                    )
                    o = o + 1
                    o = lax.cond(o >= DRAIN_AT, drain_all, lambda o: o, o)
                    return (q + 1, o)

                return lax.fori_loop(0, b1 - b0, one, qo)

            in_range = jnp.logical_and(s_loc >= 0, s_loc < cnt)
            return lax.cond(in_range, go, lambda qo: qo, qo)

        fire(start, 0, sem_a)
        qo = (jnp.int32(0), jnp.int32(0))

        def two_strips(ss, qo):
            t0 = start + 2 * ss
            t1 = t0 + 1
            t2 = t0 + 2
            fire(t1, 1, sem_b)
            wait(t0, 0, sem_a)
            qo = scan_strip(t0, 0, qo)
            fire(t2, 0, sem_a)
            wait(t1, 1, sem_b)
            qo = scan_strip(t1, 1, qo)
            return qo

        qo = lax.fori_loop(0, ns_static // 2, two_strips, qo)
        wait(start + ns_static, 0, sem_a)
        q, o = qo
        drain_all(o)

    return k


@jax.jit
def kernel(class_labels, embedding_weight):
    (B,) = class_labels.shape
    V, D = embedding_weight.shape
    k = _make_sc_gather(V, D, B)
    n_last = V % 128 if V % 128 else 128
    lastT = jnp.pad(
        embedding_weight[V - n_last :, :].T, ((0, 0), (0, 128 - n_last))
    )
    return k(class_labels.astype(jnp.int32), embedding_weight.T, lastT)


# speedup vs baseline: 1.1187x; 1.0035x over previous
import functools

import jax
import jax.numpy as jnp
from jax import lax
from jax.experimental import pallas as pl
from jax.experimental.pallas import tpu as pltpu
from jax.experimental.pallas import tpu_sc as plsc

RING = 128
DRAIN_AT = RING - 16


def _make_sc_gather(V, D, B):
    info = plsc.get_sparse_core_info()
    NC, NS = info.num_cores, info.num_subcores
    NW = NC * NS
    assert D == 64 and B % 16 == 0
    n_strips_all = (V + 127) // 128
    last_strip = V // 128
    strips_base = n_strips_all // NW
    strips_rem = n_strips_all % NW
    n_vreg_idx = B // 16
    ns_static = strips_base + 2 - ((strips_base + 2) % 2)
    smax = strips_base + 1
    mesh = plsc.VectorSubcoreMesh(core_axis_name="c", subcore_axis_name="s")

    @functools.partial(
        pl.kernel,
        mesh=mesh,
        out_type=jax.ShapeDtypeStruct((B, D), jnp.float32),
        scratch_types=[
            pltpu.VMEM((B,), jnp.int32),
            pltpu.VMEM((B + 16,), jnp.int32),
            pltpu.VMEM((B + 16,), jnp.int32),
            pltpu.VMEM((B + 16,), jnp.int32),
            pltpu.VMEM((B + 16,), jnp.int32),
            pltpu.VMEM((2, 64, 128), jnp.float32),
            pltpu.VMEM((RING, 64), jnp.float32),
            pltpu.SMEM((smax + 2,), jnp.int32),
            pltpu.SMEM((smax + 2,), jnp.int32),
            pltpu.SemaphoreType.DMA,
            pltpu.SemaphoreType.DMA,
            pltpu.SemaphoreType.DMA,
        ],
        compiler_params=pltpu.CompilerParams(needs_layout_passes=False),
    )
    def k(idx_hbm, tableT_hbm, lastT_hbm, out_hbm, idx_v, pi_v, pj_v, qi_v,
          qj_v, strips, ring2, bins_s, offs_s, sem_a, sem_b, sem_out):
        w = lax.axis_index("s") * NC + lax.axis_index("c")
        start = w * strips_base + jnp.minimum(w, strips_rem)
        cnt = strips_base + (w < strips_rem).astype(jnp.int32)
        end = start + cnt
        trash = jnp.int32(smax)

        pltpu.sync_copy(idx_hbm, idx_v)

        lane = lax.iota(jnp.int32, 16)
        neg1 = jnp.full((16,), -1, jnp.int32)

        @plsc.parallel_loop(0, n_vreg_idx, unroll=8, carry=jnp.int32(0))
        def collect(g, ptr):
            v = idx_v[pl.ds(g * 16, 16)]
            tcv = lax.shift_right_logical(v, 7)
            m = jnp.logical_and(tcv >= start, tcv < end)
            cums = plsc.cumsum(m.astype(jnp.int32))
            pos = cums + (ptr - 1)
            plsc.store_scatter(pi_v, [pos], v, mask=m)
            plsc.store_scatter(pj_v, [pos], g * 16 + lane, mask=m)
            return ptr + cums[15]

        n_w = collect
        n_pv = (n_w + 15) // 16
        plsc.store_scatter(pi_v, [n_w + lane], neg1)

        def zero_hist(s, c):
            offs_s[s] = 0
            return c

        lax.fori_loop(0, smax + 2, zero_hist, 0)

        def hist(p, c):
            iv = pi_v[pl.ds(p * 16, 16)]
            sv = lax.shift_right_logical(iv, 7) - start
            sv = jnp.clip(sv, 0, trash)
            for l in range(16):
                s = sv[l]
                offs_s[s] = offs_s[s] + 1
            return c

        lax.fori_loop(0, n_pv, hist, 0)

        def prefix(s, run):
            h = offs_s[s]
            bins_s[s] = run
            offs_s[s] = run
            return run + h

        lax.fori_loop(0, smax + 2, prefix, jnp.int32(0))

        def place(p, c):
            iv = pi_v[pl.ds(p * 16, 16)]
            jv = pj_v[pl.ds(p * 16, 16)]
            sv = lax.shift_right_logical(iv, 7) - start
            sv = jnp.clip(sv, 0, trash)
            for l in range(16):
                s = sv[l]
                pos = offs_s[s]
                offs_s[s] = pos + 1
                posv = jnp.full((16,), pos, jnp.int32)
                onehot = lane == l
                plsc.store_scatter(qi_v, [posv], iv, mask=onehot)
                plsc.store_scatter(qj_v, [posv], jv, mask=onehot)
            return c

        lax.fori_loop(0, n_pv, place, 0)

        def strip_needed(tc):
            s_loc = tc - start
            sc = jnp.clip(s_loc, 0, smax)
            nonempty = bins_s[sc + 1] > bins_s[sc]
            return jnp.logical_and(
                jnp.logical_and(s_loc >= 0, s_loc < cnt), nonempty
            )

        def fire(tc, buf, sem):
            needed = strip_needed(tc)

            @pl.when(jnp.logical_and(needed, tc < last_strip))
            def _():
                off = pl.multiple_of(tc * 128, 128)
                pltpu.async_copy(
                    tableT_hbm.at[:, pl.ds(off, 128)], strips.at[buf], sem
                )

            @pl.when(jnp.logical_and(needed, tc == last_strip))
            def _():
                pltpu.async_copy(lastT_hbm, strips.at[buf], sem)

        def wait(tc, buf, sem):
            @pl.when(strip_needed(tc))
            def _():
                pltpu.make_async_copy(
                    tableT_hbm.at[:, pl.ds(0, 128)], strips.at[buf], sem
                ).wait()

        def drain_all(o):
            def dw(_, c):
                pltpu.make_async_copy(
                    ring2.at[pl.ds(0, 1), :], out_hbm.at[pl.ds(0, 1), :],
                    sem_out,
                ).wait()
                return c

            lax.fori_loop(0, o, dw, 0)
            return jnp.int32(0)

        def scan_strip(tc, buf, qo):
            s_loc = tc - start

            def go(qo):
                b0 = bins_s[s_loc]
                b1 = bins_s[s_loc + 1]
                bufv = jnp.full((16,), buf, jnp.int32)

                def one(t, qo):
                    q, o = qo
                    pv = jnp.full((16,), b0 + t, jnp.int32)
                    i = plsc.load_gather(qi_v, [pv])[0]
                    j = plsc.load_gather(qj_v, [pv])[0]
                    col = jnp.bitwise_and(i, 127)
                    colv = jnp.full((16,), col, jnp.int32)
                    slot = lax.rem(q, jnp.int32(RING))
                    slotv = jnp.full((16,), slot, jnp.int32)
                    for g4 in range(4):
                        vals = plsc.load_gather(
                            strips, [bufv, g4 * 16 + lane, colv]
                        )
                        plsc.store_scatter(
                            ring2, [slotv, g4 * 16 + lane], vals
                        )
                    pltpu.async_copy(
                        ring2.at[pl.ds(slot, 1), :],
                        out_hbm.at[pl.ds(j, 1), :],
                        sem_out,
                    )
                    o = o + 1
                    o = lax.cond(o >= DRAIN_AT, drain_all, lambda o: o, o)
                    return (q + 1, o)

                return lax.fori_loop(0, b1 - b0, one, qo)

            in_range = jnp.logical_and(s_loc >= 0, s_loc < cnt)
            return lax.cond(in_range, go, lambda qo: qo, qo)

        fire(start, 0, sem_a)
        qo = (jnp.int32(0), jnp.int32(0))

        def two_strips(ss, qo):
            t0 = start + 2 * ss
            t1 = t0 + 1
            t2 = t0 + 2
            fire(t1, 1, sem_b)
            wait(t0, 0, sem_a)
            qo = scan_strip(t0, 0, qo)
            fire(t2, 0, sem_a)
            wait(t1, 1, sem_b)
            qo = scan_strip(t1, 1, qo)
            return qo

        qo = lax.fori_loop(0, ns_static // 2, two_strips, qo)
        wait(start + ns_static, 0, sem_a)
        q, o = qo
        drain_all(o)

    return k


@jax.jit
def kernel(class_labels, embedding_weight):
    (B,) = class_labels.shape
    V, D = embedding_weight.shape
    k = _make_sc_gather(V, D, B)
    n_last = V % 128 if V % 128 else 128
    lastT = jnp.pad(
        embedding_weight[V - n_last :, :].T, ((0, 0), (0, 128 - n_last))
    )
    return k(class_labels.astype(jnp.int32), embedding_weight.T, lastT)


# speedup vs baseline: 1.5538x; 1.3890x over previous
import functools

import jax
import jax.numpy as jnp
from jax import lax
from jax.experimental import pallas as pl
from jax.experimental.pallas import tpu as pltpu
from jax.experimental.pallas import tpu_sc as plsc

RING = 64
DRAIN_AT = RING - 16
NBUF = 4


def _make_sc_gather(V, D, B):
    info = plsc.get_sparse_core_info()
    NC, NS = info.num_cores, info.num_subcores
    NW = NC * NS
    assert D == 64 and B % 16 == 0
    n_strips_all = (V + 127) // 128
    last_strip = V // 128
    strips_base = n_strips_all // NW
    strips_rem = n_strips_all % NW
    n_vreg_idx = B // 16
    ns_static = ((strips_base + 1 + NBUF - 1) // NBUF) * NBUF
    smax = strips_base + 1
    mesh = plsc.VectorSubcoreMesh(core_axis_name="c", subcore_axis_name="s")

    @functools.partial(
        pl.kernel,
        mesh=mesh,
        out_type=jax.ShapeDtypeStruct((B, D), jnp.float32),
        scratch_types=[
            pltpu.VMEM((B,), jnp.int32),
            pltpu.VMEM((B + 16,), jnp.int32),
            pltpu.VMEM((B + 16,), jnp.int32),
            pltpu.VMEM((B + 16,), jnp.int32),
            pltpu.VMEM((B + 16,), jnp.int32),
            pltpu.VMEM((NBUF, 64, 128), jnp.float32),
            pltpu.VMEM((RING, 64), jnp.float32),
            pltpu.SMEM((smax + 2,), jnp.int32),
            pltpu.SMEM((smax + 2,), jnp.int32),
            pltpu.SemaphoreType.DMA,
            pltpu.SemaphoreType.DMA,
            pltpu.SemaphoreType.DMA,
            pltpu.SemaphoreType.DMA,
            pltpu.SemaphoreType.DMA,
        ],
        compiler_params=pltpu.CompilerParams(needs_layout_passes=False),
    )
    def k(idx_hbm, tableT_hbm, lastT_hbm, out_hbm, idx_v, pi_v, pj_v, qi_v,
          qj_v, strips, ring2, bins_s, offs_s, sem_a, sem_b, sem_c, sem_d,
          sem_out):
        sems = [sem_a, sem_b, sem_c, sem_d]
        w = lax.axis_index("s") * NC + lax.axis_index("c")
        start = w * strips_base + jnp.minimum(w, strips_rem)
        cnt = strips_base + (w < strips_rem).astype(jnp.int32)
        end = start + cnt
        trash = jnp.int32(smax)

        pltpu.sync_copy(idx_hbm, idx_v)

        lane = lax.iota(jnp.int32, 16)
        neg1 = jnp.full((16,), -1, jnp.int32)

        @plsc.parallel_loop(0, n_vreg_idx, unroll=8, carry=jnp.int32(0))
        def collect(g, ptr):
            v = idx_v[pl.ds(g * 16, 16)]
            tcv = lax.shift_right_logical(v, 7)
            m = jnp.logical_and(tcv >= start, tcv < end)
            cums = plsc.cumsum(m.astype(jnp.int32))
            pos = cums + (ptr - 1)
            plsc.store_scatter(pi_v, [pos], v, mask=m)
            plsc.store_scatter(pj_v, [pos], g * 16 + lane, mask=m)
            return ptr + cums[15]

        n_w = collect
        n_pv = (n_w + 15) // 16
        plsc.store_scatter(pi_v, [n_w + lane], neg1)

        def zero_hist(s, c):
            offs_s[s] = 0
            return c

        lax.fori_loop(0, smax + 2, zero_hist, 0)

        def hist(p, c):
            iv = pi_v[pl.ds(p * 16, 16)]
            sv = lax.shift_right_logical(iv, 7) - start
            sv = jnp.clip(sv, 0, trash)
            for l in range(16):
                s = sv[l]
                offs_s[s] = offs_s[s] + 1
            return c

        lax.fori_loop(0, n_pv, hist, 0)

        def prefix(s, run):
            h = offs_s[s]
            bins_s[s] = run
            offs_s[s] = run
            return run + h

        lax.fori_loop(0, smax + 2, prefix, jnp.int32(0))

        def place(p, c):
            iv = pi_v[pl.ds(p * 16, 16)]
            jv = pj_v[pl.ds(p * 16, 16)]
            sv = lax.shift_right_logical(iv, 7) - start
            sv = jnp.clip(sv, 0, trash)
            for l in range(16):
                s = sv[l]
                pos = offs_s[s]
                offs_s[s] = pos + 1
                posv = jnp.full((16,), pos, jnp.int32)
                onehot = lane == l
                plsc.store_scatter(qi_v, [posv], iv, mask=onehot)
                plsc.store_scatter(qj_v, [posv], jv, mask=onehot)
            return c

        lax.fori_loop(0, n_pv, place, 0)

        def strip_needed(tc):
            s_loc = tc - start
            sc = jnp.clip(s_loc, 0, smax)
            nonempty = bins_s[sc + 1] > bins_s[sc]
            return jnp.logical_and(
                jnp.logical_and(s_loc >= 0, s_loc < cnt), nonempty
            )

        def fire(tc, buf, sem):
            needed = strip_needed(tc)

            @pl.when(jnp.logical_and(needed, tc < last_strip))
            def _():
                off = pl.multiple_of(tc * 128, 128)
                pltpu.async_copy(
                    tableT_hbm.at[:, pl.ds(off, 128)], strips.at[buf], sem
                )

            @pl.when(jnp.logical_and(needed, tc == last_strip))
            def _():
                pltpu.async_copy(lastT_hbm, strips.at[buf], sem)

        def wait(tc, buf, sem):
            @pl.when(strip_needed(tc))
            def _():
                pltpu.make_async_copy(
                    tableT_hbm.at[:, pl.ds(0, 128)], strips.at[buf], sem
                ).wait()

        def drain_all(o):
            def dw(_, c):
                pltpu.make_async_copy(
                    ring2.at[pl.ds(0, 1), :], out_hbm.at[pl.ds(0, 1), :],
                    sem_out,
                ).wait()
                return c

            lax.fori_loop(0, o, dw, 0)
            return jnp.int32(0)

        def scan_strip(tc, buf, qo):
            s_loc = tc - start

            def go(qo):
                b0 = bins_s[s_loc]
                b1 = bins_s[s_loc + 1]
                bufv = jnp.full((16,), buf, jnp.int32)

                def one(t, qo):
                    q, o = qo
                    pv = jnp.full((16,), b0 + t, jnp.int32)
                    i = plsc.load_gather(qi_v, [pv])[0]
                    j = plsc.load_gather(qj_v, [pv])[0]
                    col = jnp.bitwise_and(i, 127)
                    colv = jnp.full((16,), col, jnp.int32)
                    slot = lax.rem(q, jnp.int32(RING))
                    slotv = jnp.full((16,), slot, jnp.int32)
                    for g4 in range(4):
                        vals = plsc.load_gather(
                            strips, [bufv, g4 * 16 + lane, colv]
                        )
                        plsc.store_scatter(
                            ring2, [slotv, g4 * 16 + lane], vals
                        )
                    pltpu.async_copy(
                        ring2.at[pl.ds(slot, 1), :],
                        out_hbm.at[pl.ds(j, 1), :],
                        sem_out,
                    )
                    o = o + 1
                    o = lax.cond(o >= DRAIN_AT, drain_all, lambda o: o, o)
                    return (q + 1, o)

                return lax.fori_loop(0, b1 - b0, one, qo)

            in_range = jnp.logical_and(s_loc >= 0, s_loc < cnt)
            return lax.cond(in_range, go, lambda qo: qo, qo)

        for u in range(NBUF):
            fire(start + u, u, sems[u])
        qo = (jnp.int32(0), jnp.int32(0))

        def nbuf_strips(ss, qo):
            base_t = start + NBUF * ss
            for u in range(NBUF):
                tc = base_t + u
                wait(tc, u, sems[u])
                qo = scan_strip(tc, u, qo)
                fire(tc + NBUF, u, sems[u])
            return qo

        qo = lax.fori_loop(0, ns_static // NBUF, nbuf_strips, qo)
        for u in range(NBUF):
            wait(start + ns_static + u, u, sems[u])
        q, o = qo
        drain_all(o)

    return k


@jax.jit
def kernel(class_labels, embedding_weight):
    (B,) = class_labels.shape
    V, D = embedding_weight.shape
    k = _make_sc_gather(V, D, B)
    n_last = V % 128 if V % 128 else 128
    lastT = jnp.pad(
        embedding_weight[V - n_last :, :].T, ((0, 0), (0, 128 - n_last))
    )
    return k(class_labels.astype(jnp.int32), embedding_weight.T, lastT)
